# baseline (speedup 1.0000x reference)
import functools

import jax
import jax.numpy as jnp
from jax import lax
from jax.experimental import pallas as pl
from jax.experimental.pallas import tpu as pltpu
from jax.experimental.pallas import tpu_sc as plsc

N = 10000
E = 320000
D = 128

NC = 2
NS = 16
NW = NC * NS
EW = E // NW
B = 80
NB = EW // B
CH = 80
NCH = N // CH
NV = EW // 16

_BLK = 2048
_GRID = -(-N // _BLK)
NP = _GRID * _BLK

_mesh = plsc.VectorSubcoreMesh(core_axis_name="c", subcore_axis_name="s")


@functools.partial(
    pl.kernel,
    out_type=(
        jax.ShapeDtypeStruct((_GRID, NW, _BLK), jnp.float32),
        jax.ShapeDtypeStruct((_GRID, NW, _BLK), jnp.float32),
    ),
    mesh=_mesh,
    scratch_types=[
        pltpu.VMEM((EW,), jnp.int32),
        pltpu.VMEM((EW,), jnp.int32),
        pltpu.VMEM((NP,), jnp.float32),
        pltpu.VMEM((NP,), jnp.float32),
    ],
    compiler_params=pltpu.CompilerParams(needs_layout_passes=False),
)
def _sc_degrees(src_hbm, dst_hbm, do_out, di_out, ib_s, ib_d, hist_o, hist_i):
    c = lax.axis_index("c")
    s = lax.axis_index("s")
    w = s * NC + c

    pltpu.sync_copy(src_hbm.at[pl.ds(w * EW, EW)], ib_s)
    pltpu.sync_copy(dst_hbm.at[pl.ds(w * EW, EW)], ib_d)

    def zb(i, _):
        hist_o[pl.ds(i * 16, 16)] = jnp.zeros((16,), jnp.float32)
        hist_i[pl.ds(i * 16, 16)] = jnp.zeros((16,), jnp.float32)
        return 0
    lax.fori_loop(0, NP // 16, zb, 0)

    def hb(i, _):
        idx = ib_s[pl.ds(i * 16, 16)]
        cnt, last = plsc.scan_count(idx)
        plsc.addupdate_scatter(hist_o, [idx], cnt.astype(jnp.float32),
                               mask=last)
        idx2 = ib_d[pl.ds(i * 16, 16)]
        cnt2, last2 = plsc.scan_count(idx2)
        plsc.addupdate_scatter(hist_i, [idx2], cnt2.astype(jnp.float32),
                               mask=last2)
        return 0
    lax.fori_loop(0, NV, hb, 0)

    for g in range(_GRID):
        pltpu.sync_copy(hist_o.at[pl.ds(g * _BLK, _BLK)], do_out.at[g, w])
        pltpu.sync_copy(hist_i.at[pl.ds(g * _BLK, _BLK)], di_out.at[g, w])


@functools.partial(
    pl.kernel,
    out_type=jax.ShapeDtypeStruct((NC, N, D), jnp.float32),
    mesh=_mesh,
    scratch_types=[
        pltpu.VMEM((EW,), jnp.int32),
        pltpu.VMEM((NB, B), jnp.int32),
        pltpu.VMEM((B, D), jnp.float32),
        pltpu.VMEM((B, D), jnp.float32),
        pltpu.VMEM_SHARED((N, D), jnp.float32),
        pltpu.SemaphoreType.DMA,
        pltpu.SemaphoreType.DMA,
        pltpu.SemaphoreType.DMA,
        pltpu.SemaphoreType.DMA,
    ],
)
def _sc_propagate(m_hbm, src_hbm, dst_hbm, part_out, idx_s, idx_d, rows0,
                  rows1, acc, gsem0, gsem1, ssem0, ssem1):
    c = lax.axis_index("c")
    s = lax.axis_index("s")
    w = s * NC + c

    pltpu.sync_copy(src_hbm.at[pl.ds(w * EW, EW)], idx_s)
    pltpu.sync_copy(dst_hbm.at[w], idx_d)

    def zv_all(t, _):
        i = t // (D // 16)
        j = t % (D // 16)
        rows0[i, pl.ds(j * 16, 16)] = jnp.zeros((16,), jnp.float32)
        return 0
    lax.fori_loop(0, CH * (D // 16), zv_all, 0)

    def zbody(t, _):
        k = s + t * NS
        @pl.when(k < NCH)
        def _():
            pltpu.sync_copy(rows0.at[pl.ds(0, CH)], acc.at[pl.ds(k * CH, CH)])
        return 0
    lax.fori_loop(0, (NCH + NS - 1) // NS, zbody, 0)
    plsc.subcore_barrier()

    def gidx(j):
        return idx_s.at[pl.ds(j * B, B)]

    pltpu.async_copy(m_hbm.at[gidx(0)], rows0, gsem0)

    def body(i, _):
        j0 = 2 * i
        j1 = 2 * i + 1
        pltpu.make_async_copy(m_hbm.at[gidx(j0)], rows0, gsem0).wait()
        pltpu.async_copy(rows0, acc.at[idx_d.at[j0]], ssem0, add=True)

        @pl.when(i > 0)
        def _():
            pltpu.make_async_copy(rows1, acc.at[idx_d.at[j1 - 2]],
                                  ssem1).wait()

        pltpu.async_copy(m_hbm.at[gidx(j1)], rows1, gsem1)
        pltpu.make_async_copy(m_hbm.at[gidx(j1)], rows1, gsem1).wait()
        pltpu.async_copy(rows1, acc.at[idx_d.at[j1]], ssem1, add=True)
        pltpu.make_async_copy(rows0, acc.at[idx_d.at[j0]], ssem0).wait()

        @pl.when(j0 + 2 < NB)
        def _():
            pltpu.async_copy(m_hbm.at[gidx(j0 + 2)], rows0, gsem0)

        return 0

    lax.fori_loop(0, NB // 2, body, 0)
    if NB % 2:
        jt = NB - 1
        pltpu.make_async_copy(m_hbm.at[gidx(jt)], rows0, gsem0).wait()
        pltpu.async_copy(rows0, acc.at[idx_d.at[jt]], ssem0, add=True)
        pltpu.make_async_copy(rows0, acc.at[idx_d.at[jt]], ssem0).wait()
    pltpu.make_async_copy(rows1, acc.at[idx_d.at[NB - 2]], ssem1).wait()
    plsc.subcore_barrier()

    def dbody(t, _):
        k = s + t * NS
        @pl.when(k < NCH)
        def _():
            r = k * CH
            pltpu.sync_copy(acc.at[pl.ds(r, CH)], rows0.at[pl.ds(0, CH)])
            pltpu.sync_copy(rows0.at[pl.ds(0, CH)], part_out.at[c, pl.ds(r, CH)])
        return 0
    lax.fori_loop(0, (NCH + NS - 1) // NS, dbody, 0)


def _norm(deg):
    return jnp.where(deg > 0, lax.rsqrt(deg), 0.0)


def _dense1_body(dop_ref, x_ref, m1_ref):
    deg = jnp.sum(dop_ref[0], axis=0)
    m1_ref[...] = x_ref[...] * _norm(deg)[:, None]


def _dense2_body(part_ref, dip_ref, dop_ref, w_ref, b_ref, m2_ref):
    agg = part_ref[0] + part_ref[1]
    nd = _norm(jnp.sum(dip_ref[0], axis=0))
    h = jnp.dot(agg * nd[:, None], w_ref[...],
                preferred_element_type=jnp.float32) + b_ref[...]
    h = jnp.maximum(h, 0.0)
    ns = _norm(jnp.sum(dop_ref[0], axis=0))
    m2_ref[...] = h * ns[:, None]


def _dense3_body(part_ref, dip_ref, w_ref, b_ref, out_ref):
    agg = part_ref[0] + part_ref[1]
    nd = _norm(jnp.sum(dip_ref[0], axis=0))
    h = jnp.dot(agg * nd[:, None], w_ref[...],
                preferred_element_type=jnp.float32) + b_ref[...]
    out_ref[...] = jnp.maximum(h, 0.0)


_deg_spec = pl.BlockSpec((1, NW, _BLK), lambda i: (i, 0, 0))
_row_spec = pl.BlockSpec((_BLK, D), lambda i: (i, 0))
_part_spec = pl.BlockSpec((NC, _BLK, D), lambda i: (0, i, 0))
_w_spec = pl.BlockSpec((D, D), lambda i: (0, 0))
_b_spec = pl.BlockSpec((1, D), lambda i: (0, 0))

_dense1 = pl.pallas_call(
    _dense1_body,
    grid=(_GRID,),
    in_specs=[_deg_spec, _row_spec],
    out_specs=_row_spec,
    out_shape=jax.ShapeDtypeStruct((N, D), jnp.float32),
)

_dense2 = pl.pallas_call(
    _dense2_body,
    grid=(_GRID,),
    in_specs=[_part_spec, _deg_spec, _deg_spec, _w_spec, _b_spec],
    out_specs=_row_spec,
    out_shape=jax.ShapeDtypeStruct((N, D), jnp.float32),
)

_dense3 = pl.pallas_call(
    _dense3_body,
    grid=(_GRID,),
    in_specs=[_part_spec, _deg_spec, _w_spec, _b_spec],
    out_specs=_row_spec,
    out_shape=jax.ShapeDtypeStruct((N, D), jnp.float32),
)


def kernel(features, edge_index, W1, b1, W2, b2):
    src_flat = edge_index[0]
    dst_flat = edge_index[1]
    dst = dst_flat.reshape(NW, NB, B)
    b1r = b1.reshape(1, D)
    b2r = b2.reshape(1, D)

    dop, dip = _sc_degrees(src_flat, dst_flat)
    m1 = _dense1(dop, features)
    part1 = _sc_propagate(m1, src_flat, dst)
    m2 = _dense2(part1, dip, dop, W1, b1r)
    part2 = _sc_propagate(m2, src_flat, dst)
    h2 = _dense3(part2, dip, W2, b2r)
    return h2


# speedup vs baseline: 1.0617x; 1.0617x over previous
import functools

import jax
import jax.numpy as jnp
from jax import lax
from jax.experimental import pallas as pl
from jax.experimental.pallas import tpu as pltpu
from jax.experimental.pallas import tpu_sc as plsc

N = 10000
E = 320000
D = 128

NC = 2
NS = 16
NW = NC * NS
EW = E // NW
B = 80
NB = EW // B
CH = 80
NCH = N // CH
NV = EW // 16

_BLK = 2048
_GRID = -(-N // _BLK)
NP = _GRID * _BLK

_mesh = plsc.VectorSubcoreMesh(core_axis_name="c", subcore_axis_name="s")


@functools.partial(
    pl.kernel,
    out_type=(
        jax.ShapeDtypeStruct((_GRID, NW, _BLK), jnp.float32),
        jax.ShapeDtypeStruct((_GRID, NW, _BLK), jnp.float32),
    ),
    mesh=_mesh,
    scratch_types=[
        pltpu.VMEM((EW,), jnp.int32),
        pltpu.VMEM((EW,), jnp.int32),
        pltpu.VMEM((NP,), jnp.float32),
        pltpu.VMEM((NP,), jnp.float32),
    ],
    compiler_params=pltpu.CompilerParams(needs_layout_passes=False),
)
def _sc_degrees(src_hbm, dst_hbm, do_out, di_out, ib_s, ib_d, hist_o, hist_i):
    c = lax.axis_index("c")
    s = lax.axis_index("s")
    w = s * NC + c

    pltpu.sync_copy(src_hbm.at[pl.ds(w * EW, EW)], ib_s)
    pltpu.sync_copy(dst_hbm.at[pl.ds(w * EW, EW)], ib_d)

    def zb(i, _):
        hist_o[pl.ds(i * 16, 16)] = jnp.zeros((16,), jnp.float32)
        hist_i[pl.ds(i * 16, 16)] = jnp.zeros((16,), jnp.float32)
        return 0
    lax.fori_loop(0, NP // 16, zb, 0)

    def hb(i, _):
        idx = ib_s[pl.ds(i * 16, 16)]
        cnt, last = plsc.scan_count(idx)
        plsc.addupdate_scatter(hist_o, [idx], cnt.astype(jnp.float32),
                               mask=last)
        idx2 = ib_d[pl.ds(i * 16, 16)]
        cnt2, last2 = plsc.scan_count(idx2)
        plsc.addupdate_scatter(hist_i, [idx2], cnt2.astype(jnp.float32),
                               mask=last2)
        return 0
    lax.fori_loop(0, NV, hb, 0)

    for g in range(_GRID):
        pltpu.sync_copy(hist_o.at[pl.ds(g * _BLK, _BLK)], do_out.at[g, w])
        pltpu.sync_copy(hist_i.at[pl.ds(g * _BLK, _BLK)], di_out.at[g, w])


@functools.partial(
    pl.kernel,
    out_type=jax.ShapeDtypeStruct((NC, N, D), jnp.float32),
    mesh=_mesh,
    scratch_types=[
        pltpu.VMEM((EW,), jnp.int32),
        pltpu.VMEM((NB, B), jnp.int32),
        pltpu.VMEM((B, D), jnp.float32),
        pltpu.VMEM((B, D), jnp.float32),
        pltpu.VMEM_SHARED((N, D), jnp.float32),
        pltpu.SemaphoreType.DMA,
        pltpu.SemaphoreType.DMA,
        pltpu.SemaphoreType.DMA,
        pltpu.SemaphoreType.DMA,
        pltpu.SemaphoreType.DMA,
        pltpu.SemaphoreType.DMA,
    ],
)
def _sc_propagate(m_hbm, src_hbm, dst_hbm, part_out, idx_s, idx_d, rows0,
                  rows1, acc, gsem0, gsem0b, gsem1, gsem1b, ssem0, ssem1):
    c = lax.axis_index("c")
    s = lax.axis_index("s")
    w = s * NC + c

    pltpu.sync_copy(src_hbm.at[pl.ds(w * EW, EW)], idx_s)
    pltpu.sync_copy(dst_hbm.at[w], idx_d)

    def zv_all(t, _):
        i = t // (D // 16)
        j = t % (D // 16)
        rows0[i, pl.ds(j * 16, 16)] = jnp.zeros((16,), jnp.float32)
        return 0
    lax.fori_loop(0, CH * (D // 16), zv_all, 0)

    def zbody(t, _):
        k = s + t * NS
        @pl.when(k < NCH)
        def _():
            pltpu.sync_copy(rows0.at[pl.ds(0, CH)], acc.at[pl.ds(k * CH, CH)])
        return 0
    lax.fori_loop(0, (NCH + NS - 1) // NS, zbody, 0)
    plsc.subcore_barrier()

    HB = B // 2

    def _gather(j, buf, semA, semB, start):
        iA = idx_s.at[pl.ds(j * B, HB)]
        iB = idx_s.at[pl.ds(j * B + HB, HB)]
        bA = buf.at[pl.ds(0, HB)]
        bB = buf.at[pl.ds(HB, HB)]
        if start:
            pltpu.async_copy(m_hbm.at[iA], bA, semA)
            pltpu.async_copy(m_hbm.at[iB], bB, semB)
        else:
            pltpu.make_async_copy(m_hbm.at[iA], bA, semA).wait()
            pltpu.make_async_copy(m_hbm.at[iB], bB, semB).wait()

    _gather(0, rows0, gsem0, gsem0b, True)

    def body(i, _):
        j0 = 2 * i
        j1 = 2 * i + 1
        _gather(j0, rows0, gsem0, gsem0b, False)
        pltpu.async_copy(rows0, acc.at[idx_d.at[j0]], ssem0, add=True)

        @pl.when(i > 0)
        def _():
            pltpu.make_async_copy(rows1, acc.at[idx_d.at[j1 - 2]],
                                  ssem1).wait()

        _gather(j1, rows1, gsem1, gsem1b, True)
        _gather(j1, rows1, gsem1, gsem1b, False)
        pltpu.async_copy(rows1, acc.at[idx_d.at[j1]], ssem1, add=True)
        pltpu.make_async_copy(rows0, acc.at[idx_d.at[j0]], ssem0).wait()

        @pl.when(j0 + 2 < NB)
        def _():
            _gather(j0 + 2, rows0, gsem0, gsem0b, True)

        return 0

    lax.fori_loop(0, NB // 2, body, 0)
    if NB % 2:
        jt = NB - 1
        _gather(jt, rows0, gsem0, gsem0b, False)
        pltpu.async_copy(rows0, acc.at[idx_d.at[jt]], ssem0, add=True)
        pltpu.make_async_copy(rows0, acc.at[idx_d.at[jt]], ssem0).wait()
    pltpu.make_async_copy(rows1, acc.at[idx_d.at[NB - 2]], ssem1).wait()
    plsc.subcore_barrier()

    def dbody(t, _):
        k = s + t * NS
        @pl.when(k < NCH)
        def _():
            r = k * CH
            pltpu.sync_copy(acc.at[pl.ds(r, CH)], rows0.at[pl.ds(0, CH)])
            pltpu.sync_copy(rows0.at[pl.ds(0, CH)], part_out.at[c, pl.ds(r, CH)])
        return 0
    lax.fori_loop(0, (NCH + NS - 1) // NS, dbody, 0)


def _norm(deg):
    return jnp.where(deg > 0, lax.rsqrt(deg), 0.0)


def _dense1_body(dop_ref, x_ref, m1_ref):
    deg = jnp.sum(dop_ref[0], axis=0)
    m1_ref[...] = x_ref[...] * _norm(deg)[:, None]


def _dense2_body(part_ref, dip_ref, dop_ref, w_ref, b_ref, m2_ref):
    agg = part_ref[0] + part_ref[1]
    nd = _norm(jnp.sum(dip_ref[0], axis=0))
    h = jnp.dot(agg * nd[:, None], w_ref[...],
                preferred_element_type=jnp.float32) + b_ref[...]
    h = jnp.maximum(h, 0.0)
    ns = _norm(jnp.sum(dop_ref[0], axis=0))
    m2_ref[...] = h * ns[:, None]


def _dense3_body(part_ref, dip_ref, w_ref, b_ref, out_ref):
    agg = part_ref[0] + part_ref[1]
    nd = _norm(jnp.sum(dip_ref[0], axis=0))
    h = jnp.dot(agg * nd[:, None], w_ref[...],
                preferred_element_type=jnp.float32) + b_ref[...]
    out_ref[...] = jnp.maximum(h, 0.0)


_deg_spec = pl.BlockSpec((1, NW, _BLK), lambda i: (i, 0, 0))
_row_spec = pl.BlockSpec((_BLK, D), lambda i: (i, 0))
_part_spec = pl.BlockSpec((NC, _BLK, D), lambda i: (0, i, 0))
_w_spec = pl.BlockSpec((D, D), lambda i: (0, 0))
_b_spec = pl.BlockSpec((1, D), lambda i: (0, 0))

_dense1 = pl.pallas_call(
    _dense1_body,
    grid=(_GRID,),
    in_specs=[_deg_spec, _row_spec],
    out_specs=_row_spec,
    out_shape=jax.ShapeDtypeStruct((N, D), jnp.float32),
)

_dense2 = pl.pallas_call(
    _dense2_body,
    grid=(_GRID,),
    in_specs=[_part_spec, _deg_spec, _deg_spec, _w_spec, _b_spec],
    out_specs=_row_spec,
    out_shape=jax.ShapeDtypeStruct((N, D), jnp.float32),
)

_dense3 = pl.pallas_call(
    _dense3_body,
    grid=(_GRID,),
    in_specs=[_part_spec, _deg_spec, _w_spec, _b_spec],
    out_specs=_row_spec,
    out_shape=jax.ShapeDtypeStruct((N, D), jnp.float32),
)


def kernel(features, edge_index, W1, b1, W2, b2):
    src_flat = edge_index[0]
    dst_flat = edge_index[1]
    dst = dst_flat.reshape(NW, NB, B)
    b1r = b1.reshape(1, D)
    b2r = b2.reshape(1, D)

    dop, dip = _sc_degrees(src_flat, dst_flat)
    m1 = _dense1(dop, features)
    part1 = _sc_propagate(m1, src_flat, dst)
    m2 = _dense2(part1, dip, dop, W1, b1r)
    part2 = _sc_propagate(m2, src_flat, dst)
    h2 = _dense3(part2, dip, W2, b2r)
    return h2
